# Initial kernel scaffold; baseline (speedup 1.0000x reference)
import jax
import jax.numpy as jnp
from jax.experimental import pallas as pl


def kernel(semantic_state, token_ids):
    raise NotImplementedError("write your pallas kernel here")


# speedup vs baseline: 4.1507x; 4.1507x over previous
import functools

import jax
import jax.numpy as jnp
from jax import lax
from jax.experimental import pallas as pl
from jax.experimental.pallas import tpu as pltpu
from jax.experimental.pallas import tpu_sc as plsc

_VOCAB = 1000
_VOCAB_PAD = 1024
_MIN_OCC = 2
_NC = 2
_NS = 16
_NW = _NC * _NS
_L = 16
_GRP = 128


def _sc_segment_stats(x, t):
    n, d = x.shape
    chunk = n // _NW
    ngrp = chunk // _GRP
    rows_w = _VOCAB_PAD // _NS

    mesh = plsc.VectorSubcoreMesh(core_axis_name="c", subcore_axis_name="s")

    @functools.partial(
        pl.kernel,
        out_type=(
            jax.ShapeDtypeStruct((_NC * _VOCAB_PAD, d), jnp.float32),
            jax.ShapeDtypeStruct((_NC * _VOCAB_PAD, d), jnp.float32),
            jax.ShapeDtypeStruct((_NC * _VOCAB_PAD, _L), jnp.float32),
        ),
        mesh=mesh,
        compiler_params=pltpu.CompilerParams(use_tc_tiling_on_sc=False),
        scratch_types=(
            pltpu.VMEM((ngrp, _GRP), jnp.int32),
            pltpu.VMEM((chunk, d), jnp.float32),
            pltpu.VMEM((chunk, d), jnp.float32),
            pltpu.VMEM((_GRP, _L), jnp.float32),
            pltpu.VMEM((rows_w, d), jnp.float32),
            pltpu.VMEM((rows_w, _L), jnp.float32),
            pltpu.VMEM_SHARED((_VOCAB_PAD, d), jnp.float32),
            pltpu.VMEM_SHARED((_VOCAB_PAD, d), jnp.float32),
            pltpu.VMEM_SHARED((_VOCAB_PAD, _L), jnp.float32),
        ),
    )
    def k(x_hbm, t_hbm, sum_hbm, sq_hbm, cnt_hbm,
          idx_v, xv, xsqv, ones_v, z_v, zc_v, acc_s, acc_q, acc_c):
        cid = lax.axis_index("c")
        sid = lax.axis_index("s")
        wid = cid * _NS + sid

        zero = jnp.zeros((_L,), jnp.float32)
        one = jnp.ones((_L,), jnp.float32)

        @pl.loop(0, rows_w)
        def _(r):
            zc_v[r, pl.ds(0, _L)] = zero

            @pl.loop(0, d, step=_L)
            def _(c0):
                z_v[r, pl.ds(c0, _L)] = zero

        @pl.loop(0, _GRP)
        def _(r):
            ones_v[r, pl.ds(0, _L)] = one

        vbase = sid * rows_w
        pltpu.sync_copy(z_v, acc_s.at[pl.ds(vbase, rows_w), :])
        pltpu.sync_copy(z_v, acc_q.at[pl.ds(vbase, rows_w), :])
        pltpu.sync_copy(zc_v, acc_c.at[pl.ds(vbase, rows_w), :])

        pltpu.sync_copy(t_hbm.at[pl.ds(wid * ngrp, ngrp), :], idx_v)
        pltpu.sync_copy(x_hbm.at[pl.ds(wid * chunk, chunk), :], xv)

        @pl.loop(0, chunk)
        def _(r):
            @pl.loop(0, d, step=_L)
            def _(c0):
                v = xv[r, pl.ds(c0, _L)]
                xsqv[r, pl.ds(c0, _L)] = v * v

        plsc.subcore_barrier()

        for g in range(ngrp):
            idx = idx_v.at[g]
            pltpu.sync_copy(xv.at[pl.ds(g * _GRP, _GRP), :],
                            acc_s.at[idx], add=True)
            pltpu.sync_copy(xsqv.at[pl.ds(g * _GRP, _GRP), :],
                            acc_q.at[idx], add=True)
            pltpu.sync_copy(ones_v, acc_c.at[idx], add=True)

        plsc.subcore_barrier()

        obase = cid * _VOCAB_PAD + vbase
        pltpu.sync_copy(acc_s.at[pl.ds(vbase, rows_w), :],
                        sum_hbm.at[pl.ds(obase, rows_w), :])
        pltpu.sync_copy(acc_q.at[pl.ds(vbase, rows_w), :],
                        sq_hbm.at[pl.ds(obase, rows_w), :])
        pltpu.sync_copy(acc_c.at[pl.ds(vbase, rows_w), :],
                        cnt_hbm.at[pl.ds(obase, rows_w), :])

    return k(x, t)


def _finalize_tc(sums, sumsqs, counts):
    vp = _VOCAB_PAD

    def body(s_ref, q_ref, c_ref, loss_ref, nrep_ref):
        s = s_ref[:vp, :] + s_ref[vp:, :]
        q = q_ref[:vp, :] + q_ref[vp:, :]
        c = c_ref[:vp, 0:1] + c_ref[vp:, 0:1]
        mean = s / jnp.maximum(c, 1.0)
        ss = q - c * mean * mean
        var = ss / jnp.maximum(c - 1.0, 1.0)
        var_mean = jnp.sum(var, axis=1, keepdims=True) / var.shape[1]
        repeated = c >= float(_MIN_OCC)
        nrep = jnp.sum(repeated.astype(jnp.int32))
        total = jnp.sum(jnp.where(repeated, var_mean, 0.0))
        avg = total / jnp.maximum(nrep.astype(jnp.float32), 1.0)
        loss = jnp.clip(1.0 - avg, 0.0, None)
        loss_ref[0, 0] = jnp.where(nrep > 0, loss, jnp.float32(0.0))
        nrep_ref[0, 0] = nrep

    return pl.pallas_call(
        body,
        out_shape=(
            jax.ShapeDtypeStruct((1, 1), jnp.float32),
            jax.ShapeDtypeStruct((1, 1), jnp.int32),
        ),
        out_specs=(
            pl.BlockSpec(memory_space=pltpu.SMEM),
            pl.BlockSpec(memory_space=pltpu.SMEM),
        ),
    )(sums, sumsqs, counts)


@jax.jit
def kernel(semantic_state, token_ids):
    b, t_len, d = semantic_state.shape
    n = b * t_len
    x = semantic_state.reshape(n, d)
    t = token_ids.reshape(-1).astype(jnp.int32).reshape(n // _GRP, _GRP)
    sums, sumsqs, counts = _sc_segment_stats(x, t)
    loss, nrep = _finalize_tc(sums, sumsqs, counts)
    return loss[0, 0], nrep[0, 0]


# speedup vs baseline: 4.3917x; 1.0581x over previous
import functools

import jax
import jax.numpy as jnp
from jax import lax
from jax.experimental import pallas as pl
from jax.experimental.pallas import tpu as pltpu
from jax.experimental.pallas import tpu_sc as plsc

_VOCAB = 1000
_VOCAB_PAD = 1024
_MIN_OCC = 2
_NC = 2
_NS = 16
_NW = _NC * _NS
_L = 16
_GRP = 128


def _sc_segment_stats(x, t):
    n, d = x.shape
    chunk = n // _NW
    ngrp = chunk // _GRP
    rows_w = _VOCAB_PAD // _NS

    mesh = plsc.VectorSubcoreMesh(core_axis_name="c", subcore_axis_name="s")

    @functools.partial(
        pl.kernel,
        out_type=(
            jax.ShapeDtypeStruct((_NC * _VOCAB_PAD, d), jnp.float32),
            jax.ShapeDtypeStruct((_NC * _VOCAB_PAD, d), jnp.float32),
            jax.ShapeDtypeStruct((_NC * _VOCAB_PAD, _L), jnp.float32),
        ),
        mesh=mesh,
        compiler_params=pltpu.CompilerParams(use_tc_tiling_on_sc=False),
        scratch_types=(
            pltpu.VMEM((ngrp, _GRP), jnp.int32),
            pltpu.VMEM((chunk, d), jnp.float32),
            pltpu.VMEM((chunk, d), jnp.float32),
            pltpu.VMEM((_GRP, _L), jnp.float32),
            pltpu.VMEM((rows_w, d), jnp.float32),
            pltpu.VMEM((rows_w, _L), jnp.float32),
            pltpu.VMEM_SHARED((_VOCAB_PAD, d), jnp.float32),
            pltpu.VMEM_SHARED((_VOCAB_PAD, d), jnp.float32),
            pltpu.VMEM_SHARED((_VOCAB_PAD, _L), jnp.float32),
            pltpu.SemaphoreType.DMA,
            pltpu.SemaphoreType.DMA,
        ),
    )
    def k(x_hbm, t_hbm, sum_hbm, sq_hbm, cnt_hbm,
          idx_v, xv, xsqv, ones_v, z_v, zc_v, acc_s, acc_q, acc_c,
          in_sem, add_sem):
        cid = lax.axis_index("c")
        sid = lax.axis_index("s")
        wid = cid * _NS + sid

        in_t = pltpu.async_copy(t_hbm.at[pl.ds(wid * ngrp, ngrp), :],
                                idx_v, in_sem)
        in_x = pltpu.async_copy(x_hbm.at[pl.ds(wid * chunk, chunk), :],
                                xv, in_sem)

        zero = jnp.zeros((_L,), jnp.float32)
        one = jnp.ones((_L,), jnp.float32)

        @pl.loop(0, rows_w)
        def _(r):
            zc_v[r, pl.ds(0, _L)] = zero

            @pl.loop(0, d, step=_L)
            def _(c0):
                z_v[r, pl.ds(c0, _L)] = zero

        @pl.loop(0, _GRP)
        def _(r):
            ones_v[r, pl.ds(0, _L)] = one

        vbase = sid * rows_w
        z0 = pltpu.async_copy(z_v, acc_s.at[pl.ds(vbase, rows_w), :], add_sem)
        z1 = pltpu.async_copy(z_v, acc_q.at[pl.ds(vbase, rows_w), :], add_sem)
        z2 = pltpu.async_copy(zc_v, acc_c.at[pl.ds(vbase, rows_w), :], add_sem)

        in_x.wait()

        @pl.loop(0, chunk)
        def _(r):
            @pl.loop(0, d, step=_L)
            def _(c0):
                v = xv[r, pl.ds(c0, _L)]
                xsqv[r, pl.ds(c0, _L)] = v * v

        in_t.wait()
        z0.wait()
        z1.wait()
        z2.wait()
        plsc.subcore_barrier()

        adds = []
        for g in range(ngrp):
            idx = idx_v.at[g]
            adds.append(pltpu.async_copy(xv.at[pl.ds(g * _GRP, _GRP), :],
                                         acc_s.at[idx], add_sem, add=True))
            adds.append(pltpu.async_copy(xsqv.at[pl.ds(g * _GRP, _GRP), :],
                                         acc_q.at[idx], add_sem, add=True))
            adds.append(pltpu.async_copy(ones_v, acc_c.at[idx], add_sem,
                                         add=True))
        for a in adds:
            a.wait()

        plsc.subcore_barrier()

        obase = cid * _VOCAB_PAD + vbase
        w0 = pltpu.async_copy(acc_s.at[pl.ds(vbase, rows_w), :],
                              sum_hbm.at[pl.ds(obase, rows_w), :], add_sem)
        w1 = pltpu.async_copy(acc_q.at[pl.ds(vbase, rows_w), :],
                              sq_hbm.at[pl.ds(obase, rows_w), :], add_sem)
        w2 = pltpu.async_copy(acc_c.at[pl.ds(vbase, rows_w), :],
                              cnt_hbm.at[pl.ds(obase, rows_w), :], add_sem)
        w0.wait()
        w1.wait()
        w2.wait()

    return k(x, t)


def _finalize_tc(sums, sumsqs, counts):
    vp = _VOCAB_PAD

    def body(s_ref, q_ref, c_ref, loss_ref, nrep_ref):
        s = s_ref[:vp, :] + s_ref[vp:, :]
        q = q_ref[:vp, :] + q_ref[vp:, :]
        c = c_ref[:vp, 0:1] + c_ref[vp:, 0:1]
        mean = s / jnp.maximum(c, 1.0)
        ss = q - c * mean * mean
        var = ss / jnp.maximum(c - 1.0, 1.0)
        var_mean = jnp.sum(var, axis=1, keepdims=True) / var.shape[1]
        repeated = c >= float(_MIN_OCC)
        nrep = jnp.sum(repeated.astype(jnp.int32))
        total = jnp.sum(jnp.where(repeated, var_mean, 0.0))
        avg = total / jnp.maximum(nrep.astype(jnp.float32), 1.0)
        loss = jnp.clip(1.0 - avg, 0.0, None)
        loss_ref[0, 0] = jnp.where(nrep > 0, loss, jnp.float32(0.0))
        nrep_ref[0, 0] = nrep

    return pl.pallas_call(
        body,
        out_shape=(
            jax.ShapeDtypeStruct((1, 1), jnp.float32),
            jax.ShapeDtypeStruct((1, 1), jnp.int32),
        ),
        out_specs=(
            pl.BlockSpec(memory_space=pltpu.SMEM),
            pl.BlockSpec(memory_space=pltpu.SMEM),
        ),
    )(sums, sumsqs, counts)


@jax.jit
def kernel(semantic_state, token_ids):
    b, t_len, d = semantic_state.shape
    n = b * t_len
    x = semantic_state.reshape(n, d)
    t = token_ids.reshape(-1).astype(jnp.int32).reshape(n // _GRP, _GRP)
    sums, sumsqs, counts = _sc_segment_stats(x, t)
    loss, nrep = _finalize_tc(sums, sumsqs, counts)
    return loss[0, 0], nrep[0, 0]
